# Initial kernel scaffold; baseline (speedup 1.0000x reference)
import jax
import jax.numpy as jnp
from jax.experimental import pallas as pl


def kernel(x, W_gate, expert_bias, W_up, b_up, A, B):
    raise NotImplementedError("write your pallas kernel here")


# speedup vs baseline: 2.7698x; 2.7698x over previous
import jax
import jax.numpy as jnp
from jax.experimental import pallas as pl
from jax.experimental.pallas import tpu as pltpu

E = 8
R = 8
SCALE = 1.0

TM = 512
BN = 1024


def _moe_lora_kernel(x_ref, wg_ref, eb_ref, wut_ref, bu_ref, acat_ref,
                     bcat_ref, out_ref, u_scr):
    n = pl.program_id(0)
    t = pl.program_id(1)

    @pl.when(n == 0)
    def _():
        xb = x_ref[...]
        g = jax.lax.dot_general(xb, wg_ref[...], (((1,), (1,)), ((), ())),
                                preferred_element_type=jnp.float32)
        g = g + eb_ref[...]
        probs = jax.nn.softmax(g, axis=-1)
        top1 = jnp.argmax(probs, axis=-1)[:, None]
        u = jnp.dot(xb, acat_ref[...],
                    preferred_element_type=jnp.float32)
        lane = jax.lax.broadcasted_iota(jnp.int32, (TM, E * R), 1) // R
        mask = (lane == top1).astype(jnp.float32)
        u_scr[pl.ds(t * TM, TM), :] = u * (mask * SCALE)

    base = jnp.dot(x_ref[...], wut_ref[...],
                   preferred_element_type=jnp.float32)
    delta = jnp.dot(u_scr[pl.ds(t * TM, TM), :], bcat_ref[...],
                    preferred_element_type=jnp.float32)
    out_ref[...] = base + bu_ref[...] + delta


def kernel(x, W_gate, expert_bias, W_up, b_up, A, B):
    Bb, T, H = x.shape
    NT = Bb * T
    x_flat = x.reshape(NT, H)
    W_upT = W_up.T
    A_cat = A.transpose(1, 0, 2).reshape(H, E * R)
    B_cat = B.reshape(E * R, H)
    eb = expert_bias.reshape(1, E)
    bu = b_up.reshape(1, H)
    NB = H // BN
    TB = NT // TM

    out = pl.pallas_call(
        _moe_lora_kernel,
        grid=(NB, TB),
        in_specs=[
            pl.BlockSpec((TM, H), lambda n, t: (t, 0)),
            pl.BlockSpec((E, H), lambda n, t: (0, 0)),
            pl.BlockSpec((1, E), lambda n, t: (0, 0)),
            pl.BlockSpec((H, BN), lambda n, t: (0, n)),
            pl.BlockSpec((1, BN), lambda n, t: (0, n)),
            pl.BlockSpec((H, E * R), lambda n, t: (0, 0)),
            pl.BlockSpec((E * R, BN), lambda n, t: (0, n)),
        ],
        out_specs=pl.BlockSpec((TM, BN), lambda n, t: (t, n)),
        out_shape=jax.ShapeDtypeStruct((NT, H), jnp.float32),
        scratch_shapes=[pltpu.VMEM((NT, E * R), jnp.float32)],
    )(x_flat, W_gate, eb, W_upT, bu, A_cat, B_cat)
    return out.reshape(Bb, T, H)


# speedup vs baseline: 3.0200x; 1.0903x over previous
import jax
import jax.numpy as jnp
from jax.experimental import pallas as pl
from jax.experimental.pallas import tpu as pltpu

E = 8
R = 8
SCALE = 1.0

TM = 512
BN = 1024


def _moe_lora_kernel(x_ref, wg_ref, eb_ref, wut_ref, bu_ref, acat_ref,
                     bcat_ref, out_ref, u_scr):
    n = pl.program_id(0)
    t = pl.program_id(1)

    @pl.when(n == 0)
    def _():
        xb = x_ref[...]
        g = jax.lax.dot_general(xb, wg_ref[...], (((1,), (1,)), ((), ())),
                                preferred_element_type=jnp.float32)
        g = g + eb_ref[...]
        probs = jax.nn.softmax(g, axis=-1)
        top1 = jnp.argmax(probs, axis=-1)[:, None]
        u = jnp.dot(xb, acat_ref[...],
                    preferred_element_type=jnp.float32)
        lane = jax.lax.broadcasted_iota(jnp.int32, (TM, E * R), 1) // R
        mask = (lane == top1).astype(jnp.float32)
        u_scr[pl.ds(t * TM, TM), :] = u * (mask * SCALE)

    base = jnp.dot(x_ref[...].astype(jnp.bfloat16), wut_ref[...],
                   preferred_element_type=jnp.float32)
    delta = jnp.dot(u_scr[pl.ds(t * TM, TM), :], bcat_ref[...],
                    preferred_element_type=jnp.float32)
    out_ref[...] = base + bu_ref[...] + delta


def kernel(x, W_gate, expert_bias, W_up, b_up, A, B):
    Bb, T, H = x.shape
    NT = Bb * T
    x_flat = x.reshape(NT, H)
    W_upT = W_up.T.astype(jnp.bfloat16)
    A_cat = A.transpose(1, 0, 2).reshape(H, E * R)
    B_cat = B.reshape(E * R, H)
    eb = expert_bias.reshape(1, E)
    bu = b_up.reshape(1, H)
    NB = H // BN
    TB = NT // TM

    out = pl.pallas_call(
        _moe_lora_kernel,
        grid=(NB, TB),
        in_specs=[
            pl.BlockSpec((TM, H), lambda n, t: (t, 0)),
            pl.BlockSpec((E, H), lambda n, t: (0, 0)),
            pl.BlockSpec((1, E), lambda n, t: (0, 0)),
            pl.BlockSpec((H, BN), lambda n, t: (0, n)),
            pl.BlockSpec((1, BN), lambda n, t: (0, n)),
            pl.BlockSpec((H, E * R), lambda n, t: (0, 0)),
            pl.BlockSpec((E * R, BN), lambda n, t: (0, n)),
        ],
        out_specs=pl.BlockSpec((TM, BN), lambda n, t: (t, n)),
        out_shape=jax.ShapeDtypeStruct((NT, H), jnp.float32),
        scratch_shapes=[pltpu.VMEM((NT, E * R), jnp.float32)],
    )(x_flat, W_gate, eb, W_upT, bu, A_cat, B_cat)
    return out.reshape(Bb, T, H)


# speedup vs baseline: 3.0407x; 1.0069x over previous
import jax
import jax.numpy as jnp
from jax.experimental import pallas as pl
from jax.experimental.pallas import tpu as pltpu

E = 8
R = 8
SCALE = 1.0

TM = 256


def _moe_lora_kernel(x_ref, wg_ref, eb_ref, wut_ref, bu_ref, acat_ref,
                     bcat_ref, out_ref):
    xb = x_ref[...]
    g = jax.lax.dot_general(xb, wg_ref[...], (((1,), (1,)), ((), ())),
                            preferred_element_type=jnp.float32)
    g = g + eb_ref[...]
    probs = jax.nn.softmax(g, axis=-1)
    top1 = jnp.argmax(probs, axis=-1)[:, None]
    u = jnp.dot(xb, acat_ref[...],
                preferred_element_type=jnp.float32)
    lane = jax.lax.broadcasted_iota(jnp.int32, (TM, E * R), 1) // R
    mask = (lane == top1).astype(jnp.float32)
    u_masked = u * (mask * SCALE)
    base = jnp.dot(xb.astype(jnp.bfloat16), wut_ref[...],
                   preferred_element_type=jnp.float32)
    delta = jnp.dot(u_masked, bcat_ref[...],
                    preferred_element_type=jnp.float32)
    out_ref[...] = base + bu_ref[...] + delta


def kernel(x, W_gate, expert_bias, W_up, b_up, A, B):
    Bb, T, H = x.shape
    NT = Bb * T
    x_flat = x.reshape(NT, H)
    W_upT = W_up.T.astype(jnp.bfloat16)
    A_cat = A.transpose(1, 0, 2).reshape(H, E * R)
    B_cat = B.reshape(E * R, H)
    eb = expert_bias.reshape(1, E)
    bu = b_up.reshape(1, H)
    TB = NT // TM

    out = pl.pallas_call(
        _moe_lora_kernel,
        grid=(TB,),
        in_specs=[
            pl.BlockSpec((TM, H), lambda t: (t, 0)),
            pl.BlockSpec((E, H), lambda t: (0, 0)),
            pl.BlockSpec((1, E), lambda t: (0, 0)),
            pl.BlockSpec((H, H), lambda t: (0, 0)),
            pl.BlockSpec((1, H), lambda t: (0, 0)),
            pl.BlockSpec((H, E * R), lambda t: (0, 0)),
            pl.BlockSpec((E * R, H), lambda t: (0, 0)),
        ],
        out_specs=pl.BlockSpec((TM, H), lambda t: (t, 0)),
        out_shape=jax.ShapeDtypeStruct((NT, H), jnp.float32),
    )(x_flat, W_gate, eb, W_upT, bu, A_cat, B_cat)
    return out.reshape(Bb, T, H)
